# Initial kernel scaffold; baseline (speedup 1.0000x reference)
import jax
import jax.numpy as jnp
from jax.experimental import pallas as pl


def kernel(x, edge_index, edge_weight, roi_num, batch, device, W1, b1, W11, b11, W2, b2, w4, b4, Wl1, bl1, Wl3, bl3, Wl11, bl11, Wl33, bl33, Wl4, bl4, Wl5, bl5, Wl6, bl6, Wl7, bl7):
    raise NotImplementedError("write your pallas kernel here")


# speedup vs baseline: 137.1624x; 137.1624x over previous
import functools

import jax
import jax.numpy as jnp
from jax import lax
from jax.experimental import pallas as pl
from jax.experimental.pallas import tpu as pltpu

ROI = 116
LENN = 6670
B = 128
SEG2 = 2 * LENN + ROI
HID = 64
F32 = jnp.float32

_DN = lambda c_lhs, c_rhs: ((c_lhs, c_rhs), ((), ()))


def _dot(a, b, dn=(((1,), (0,)), ((), ()))):
    return lax.dot_general(a, b, dn, preferred_element_type=F32)


def _build_body(srcf_ref, dstf_ref, wf_ref, srcs_ref, dsts_ref, ws_ref,
                af_ref, as_ref):
    n_iota = lax.broadcasted_iota(jnp.int32, (ROI, 1), 0)

    def one(src_ref, dst_ref, w_ref):
        src = src_ref[0]
        dst = dst_ref[0]
        w = w_ref[0]
        oh_s = (src == n_iota).astype(F32)
        oh_d = (dst == n_iota).astype(F32)
        return _dot(oh_d * w, oh_s, _DN((1,), (1,)))

    af_ref[0] = one(srcf_ref, dstf_ref, wf_ref)
    as_ref[0] = one(srcs_ref, dsts_ref, ws_ref)


def _z_body(xf_ref, xs_ref, af_ref, as_ref, W1_ref, b1_ref, W11_ref, b11_ref,
            w4_ref, b4_ref, z1_ref, z2_ref, ap_ref):
    ones = jnp.full((ROI, 1), 1.0, dtype=F32)

    def gcn(x, A, W, b):
        h = _dot(x, W)
        deg = _dot(A, ones) + 1.0
        dinv = lax.rsqrt(deg)
        out = dinv * _dot(A, dinv * h) + (dinv * dinv) * h + b
        return jnp.maximum(out, 0.0)

    z1 = gcn(xf_ref[0], af_ref[0], W1_ref[...], b1_ref[...])
    z2 = gcn(xs_ref[0], as_ref[0], W11_ref[...], b11_ref[...])
    z1_ref[0] = z1
    z2_ref[0] = z2
    w4 = w4_ref[...]
    ap = (_dot(z1, w4[:, :HID], _DN((1,), (1,))) +
          _dot(z2, w4[:, HID:], _DN((1,), (1,))) + b4_ref[0, 0])
    ap_ref[0] = ap


def _softmax(l):
    m = jnp.max(l, axis=1, keepdims=True)
    e = jnp.exp(l - m)
    return e / jnp.sum(e, axis=1, keepdims=True)


def _head_body(z1f_ref, z2f_ref, ap_ref,
               Wl1_ref, bl1_ref, Wl3_ref, bl3_ref,
               Wl11_ref, bl11_ref, Wl33_ref, bl33_ref,
               Wl4_ref, bl4_ref, Wl5_ref, bl5_ref,
               x1_ref, x2_ref, alls_ref):
    def mlp_head(zf, Wa, ba, Wb, bb):
        t = jnp.maximum(_dot(zf, Wa) + ba, 0.0)
        return _softmax(_dot(t, Wb) + bb)

    x1_ref[...] = mlp_head(z1f_ref[...], Wl1_ref[...], bl1_ref[...],
                           Wl3_ref[...], bl3_ref[...])
    x2_ref[...] = mlp_head(z2f_ref[...], Wl11_ref[...], bl11_ref[...],
                           Wl33_ref[...], bl33_ref[...])
    a = jnp.maximum(_dot(ap_ref[...], Wl4_ref[...]) + bl4_ref[...], 0.0)
    a = jnp.maximum(_dot(a, Wl5_ref[...]) + bl5_ref[...], 0.0)
    alls_ref[...] = a


def _g2_body(z1_ref, z2_ref, al_ref, af_ref, as_ref, W2_ref, b2_ref, xc_ref):
    ones = jnp.full((ROI, 1), 1.0, dtype=F32)
    Af = af_ref[0]
    As = as_ref[0]
    al = al_ref[0]
    b2 = b2_ref[...]
    ht = _dot(z1_ref[0], W2_ref[...])
    hb = _dot(z2_ref[0], W2_ref[...])
    degt = _dot(Af, ones) + 1.0
    degb = _dot(As, ones) + al + 1.0
    dt = lax.rsqrt(degt)
    db = lax.rsqrt(degb)
    ot = dt * _dot(Af, dt * ht) + (dt * dt) * ht + b2
    ob = (db * _dot(As, db * hb) + (db * db) * hb +
          (db * al * dt) * ht + b2)
    xc_ref[0] = jnp.maximum(jnp.concatenate([ot, ob], axis=0), 0.0)


def _final_body(xcf_ref, Wl6_ref, bl6_ref, Wl7_ref, bl7_ref, xf_ref):
    t = jnp.maximum(_dot(xcf_ref[...], Wl6_ref[...]) + bl6_ref[...], 0.0)
    xf_ref[...] = _softmax(_dot(t, Wl7_ref[...]) + bl7_ref[...])


def _g_spec(*blk):
    return pl.BlockSpec((1,) + blk, lambda g: (g,) + (0,) * len(blk))


def _w_spec(shape):
    nd = len(shape)
    return pl.BlockSpec(shape, lambda g: (0,) * nd)


def kernel(x, edge_index, edge_weight, roi_num, batch, device,
           W1, b1, W11, b11, W2, b2, w4, b4, Wl1, bl1, Wl3, bl3,
           Wl11, bl11, Wl33, bl33, Wl4, bl4, Wl5, bl5, Wl6, bl6, Wl7, bl7):
    del roi_num, batch, device
    ei = edge_index.astype(jnp.int32).reshape(2, B, SEG2)
    base = (jnp.arange(B, dtype=jnp.int32) * (2 * ROI))[None, :, None]
    loc = ei - base
    srcf = loc[0, :, :LENN].reshape(B, 1, LENN)
    dstf = loc[1, :, :LENN].reshape(B, 1, LENN)
    srcs = (loc[0, :, LENN:2 * LENN] - ROI).reshape(B, 1, LENN)
    dsts = (loc[1, :, LENN:2 * LENN] - ROI).reshape(B, 1, LENN)
    ew_r = edge_weight.reshape(B, 1, SEG2)
    wf = ew_r[:, :, :LENN]
    ws = ew_r[:, :, LENN:2 * LENN]
    x3 = x.reshape(B, 2 * ROI, ROI - 1)
    xf3 = x3[:, :ROI]
    xs3 = x3[:, ROI:]
    b1r = b1.reshape(1, HID)
    b11r = b11.reshape(1, HID)
    b2r = b2.reshape(1, HID)
    w4r = w4.reshape(1, 2 * HID)
    b4r = b4.reshape(1, 1)
    bl1r, bl3r = bl1.reshape(1, -1), bl3.reshape(1, -1)
    bl11r, bl33r = bl11.reshape(1, -1), bl33.reshape(1, -1)
    bl4r, bl5r = bl4.reshape(1, -1), bl5.reshape(1, -1)
    bl6r, bl7r = bl6.reshape(1, -1), bl7.reshape(1, -1)

    Af, As = pl.pallas_call(
        _build_body,
        grid=(B,),
        in_specs=[_g_spec(1, LENN)] * 6,
        out_specs=[_g_spec(ROI, ROI)] * 2,
        out_shape=[jax.ShapeDtypeStruct((B, ROI, ROI), F32)] * 2,
    )(srcf, dstf, wf, srcs, dsts, ws)

    z1, z2, ap = pl.pallas_call(
        _z_body,
        grid=(B,),
        in_specs=[_g_spec(ROI, ROI - 1), _g_spec(ROI, ROI - 1),
                  _g_spec(ROI, ROI), _g_spec(ROI, ROI),
                  _w_spec((ROI - 1, HID)), _w_spec((1, HID)),
                  _w_spec((ROI - 1, HID)), _w_spec((1, HID)),
                  _w_spec((1, 2 * HID)), _w_spec((1, 1))],
        out_specs=[_g_spec(ROI, HID), _g_spec(ROI, HID), _g_spec(ROI, 1)],
        out_shape=[jax.ShapeDtypeStruct((B, ROI, HID), F32),
                   jax.ShapeDtypeStruct((B, ROI, HID), F32),
                   jax.ShapeDtypeStruct((B, ROI, 1), F32)],
    )(xf3, xs3, Af, As, W1, b1r, W11, b11r, w4r, b4r)

    z1f = z1.reshape(B, ROI * HID)
    z2f = z2.reshape(B, ROI * HID)
    apf = ap.reshape(B, ROI)
    x1, x2, alls = pl.pallas_call(
        _head_body,
        out_shape=[jax.ShapeDtypeStruct((B, 2), F32),
                   jax.ShapeDtypeStruct((B, 2), F32),
                   jax.ShapeDtypeStruct((B, ROI), F32)],
    )(z1f, z2f, apf, Wl1, bl1r, Wl3, bl3r, Wl11, bl11r, Wl33, bl33r,
      Wl4, bl4r, Wl5, bl5r)

    al3 = alls.reshape(B, ROI, 1)
    xc = pl.pallas_call(
        _g2_body,
        grid=(B,),
        in_specs=[_g_spec(ROI, HID), _g_spec(ROI, HID), _g_spec(ROI, 1),
                  _g_spec(ROI, ROI), _g_spec(ROI, ROI),
                  _w_spec((HID, HID)), _w_spec((1, HID))],
        out_specs=_g_spec(2 * ROI, HID),
        out_shape=jax.ShapeDtypeStruct((B, 2 * ROI, HID), F32),
    )(z1, z2, al3, Af, As, W2, b2r)

    xcf = xc.reshape(B, 2 * ROI * HID)
    xf = pl.pallas_call(
        _final_body,
        out_shape=jax.ShapeDtypeStruct((B, 2), F32),
    )(xcf, Wl6, bl6r, Wl7, bl7r)

    return (xf, x1, x2, alls)


# speedup vs baseline: 169.8821x; 1.2385x over previous
import functools

import jax
import jax.numpy as jnp
from jax import lax
from jax.experimental import pallas as pl
from jax.experimental.pallas import tpu as pltpu
from jax.experimental.pallas import tpu_sc as plsc

ROI = 116
LENN = 6670
B = 128
SEG2 = 2 * LENN + ROI
HID = 64
F32 = jnp.float32

_NC = 2
_NS = 16
_NW = _NC * _NS
_NMAT = 2 * B
_MPW = _NMAT // _NW
_EPAD = 6672
_AFLAT = ROI * ROI
_ACHUNKS = _AFLAT // 16
_ECHUNKS = _EPAD // 16

_DN = lambda c_lhs, c_rhs: ((c_lhs, c_rhs), ((), ()))


def _dot(a, b, dn=(((1,), (0,)), ((), ()))):
    return lax.dot_general(a, b, dn, preferred_element_type=F32)


def _build_body(srcf_ref, dstf_ref, wf_ref, srcs_ref, dsts_ref, ws_ref,
                af_ref, as_ref):
    n_iota = lax.broadcasted_iota(jnp.int32, (ROI, 1), 0)

    def one(src_ref, dst_ref, w_ref):
        src = src_ref[0]
        dst = dst_ref[0]
        w = w_ref[0]
        oh_s = (src == n_iota).astype(F32)
        oh_d = (dst == n_iota).astype(F32)
        return _dot(oh_d * w, oh_s, _DN((1,), (1,)))

    af_ref[0] = one(srcf_ref, dstf_ref, wf_ref)
    as_ref[0] = one(srcs_ref, dsts_ref, ws_ref)


def _sc_build_body(idx_hbm, w_hbm, out_hbm, idx_v, w_v, acc_v):
    wid = lax.axis_index("s") * _NC + lax.axis_index("c")

    def do_mat(i, carry):
        r = wid * _MPW + i
        pltpu.sync_copy(idx_hbm.at[r], idx_v)
        pltpu.sync_copy(w_hbm.at[r], w_v)

        def zero(j, c):
            acc_v[pl.ds(j * 16, 16)] = jnp.zeros((16,), F32)
            return c

        lax.fori_loop(0, _ACHUNKS, zero, 0, unroll=8)

        def scat(k, c):
            iv = idx_v[pl.ds(k * 16, 16)]
            wv = w_v[pl.ds(k * 16, 16)]
            plsc.addupdate_scatter(acc_v, [iv], wv)
            return c

        lax.fori_loop(0, _ECHUNKS, scat, 0, unroll=8)
        pltpu.sync_copy(acc_v, out_hbm.at[r])
        return carry

    lax.fori_loop(0, _MPW, do_mat, 0)


def _sc_build(idx_all, w_all):
    fn = pl.kernel(
        _sc_build_body,
        out_type=jax.ShapeDtypeStruct((_NMAT, _AFLAT), F32),
        mesh=plsc.VectorSubcoreMesh(core_axis_name="c", subcore_axis_name="s"),
        compiler_params=pltpu.CompilerParams(needs_layout_passes=False),
        scratch_types=[
            pltpu.VMEM((_EPAD,), jnp.int32),
            pltpu.VMEM((_EPAD,), F32),
            pltpu.VMEM((_AFLAT,), F32),
        ],
    )
    return fn(idx_all, w_all)


def _z_body(xf_ref, xs_ref, af_ref, as_ref, W1_ref, b1_ref, W11_ref, b11_ref,
            w4_ref, b4_ref, z1_ref, z2_ref, ap_ref):
    ones = jnp.full((ROI, 1), 1.0, dtype=F32)

    def gcn(x, A, W, b):
        h = _dot(x, W)
        deg = _dot(A, ones) + 1.0
        dinv = lax.rsqrt(deg)
        out = dinv * _dot(A, dinv * h) + (dinv * dinv) * h + b
        return jnp.maximum(out, 0.0)

    z1 = gcn(xf_ref[0], af_ref[0], W1_ref[...], b1_ref[...])
    z2 = gcn(xs_ref[0], as_ref[0], W11_ref[...], b11_ref[...])
    z1_ref[0] = z1
    z2_ref[0] = z2
    w4 = w4_ref[...]
    ap = (_dot(z1, w4[:, :HID], _DN((1,), (1,))) +
          _dot(z2, w4[:, HID:], _DN((1,), (1,))) + b4_ref[0, 0])
    ap_ref[0] = ap


def _softmax(l):
    m = jnp.max(l, axis=1, keepdims=True)
    e = jnp.exp(l - m)
    return e / jnp.sum(e, axis=1, keepdims=True)


def _head_body(z1f_ref, z2f_ref, ap_ref,
               Wl1_ref, bl1_ref, Wl3_ref, bl3_ref,
               Wl11_ref, bl11_ref, Wl33_ref, bl33_ref,
               Wl4_ref, bl4_ref, Wl5_ref, bl5_ref,
               x1_ref, x2_ref, alls_ref):
    def mlp_head(zf, Wa, ba, Wb, bb):
        t = jnp.maximum(_dot(zf, Wa) + ba, 0.0)
        return _softmax(_dot(t, Wb) + bb)

    x1_ref[...] = mlp_head(z1f_ref[...], Wl1_ref[...], bl1_ref[...],
                           Wl3_ref[...], bl3_ref[...])
    x2_ref[...] = mlp_head(z2f_ref[...], Wl11_ref[...], bl11_ref[...],
                           Wl33_ref[...], bl33_ref[...])
    a = jnp.maximum(_dot(ap_ref[...], Wl4_ref[...]) + bl4_ref[...], 0.0)
    a = jnp.maximum(_dot(a, Wl5_ref[...]) + bl5_ref[...], 0.0)
    alls_ref[...] = a


def _g2_body(z1_ref, z2_ref, al_ref, af_ref, as_ref, W2_ref, b2_ref, xc_ref):
    ones = jnp.full((ROI, 1), 1.0, dtype=F32)
    Af = af_ref[0]
    As = as_ref[0]
    al = al_ref[0]
    b2 = b2_ref[...]
    ht = _dot(z1_ref[0], W2_ref[...])
    hb = _dot(z2_ref[0], W2_ref[...])
    degt = _dot(Af, ones) + 1.0
    degb = _dot(As, ones) + al + 1.0
    dt = lax.rsqrt(degt)
    db = lax.rsqrt(degb)
    ot = dt * _dot(Af, dt * ht) + (dt * dt) * ht + b2
    ob = (db * _dot(As, db * hb) + (db * db) * hb +
          (db * al * dt) * ht + b2)
    xc_ref[0] = jnp.maximum(jnp.concatenate([ot, ob], axis=0), 0.0)


def _final_body(xcf_ref, Wl6_ref, bl6_ref, Wl7_ref, bl7_ref, xf_ref):
    t = jnp.maximum(_dot(xcf_ref[...], Wl6_ref[...]) + bl6_ref[...], 0.0)
    xf_ref[...] = _softmax(_dot(t, Wl7_ref[...]) + bl7_ref[...])


def _g_spec(*blk):
    return pl.BlockSpec((1,) + blk, lambda g: (g,) + (0,) * len(blk))


def _w_spec(shape):
    nd = len(shape)
    return pl.BlockSpec(shape, lambda g: (0,) * nd)


def kernel(x, edge_index, edge_weight, roi_num, batch, device,
           W1, b1, W11, b11, W2, b2, w4, b4, Wl1, bl1, Wl3, bl3,
           Wl11, bl11, Wl33, bl33, Wl4, bl4, Wl5, bl5, Wl6, bl6, Wl7, bl7):
    del roi_num, batch, device
    ei = edge_index.astype(jnp.int32).reshape(2, B, SEG2)
    base = (jnp.arange(B, dtype=jnp.int32) * (2 * ROI))[None, :, None]
    loc = ei - base
    srcf = loc[0, :, :LENN]
    dstf = loc[1, :, :LENN]
    srcs = loc[0, :, LENN:2 * LENN] - ROI
    dsts = loc[1, :, LENN:2 * LENN] - ROI
    ew_r = edge_weight.reshape(B, SEG2)
    wf = ew_r[:, :LENN]
    ws = ew_r[:, LENN:2 * LENN]
    idx_all = jnp.concatenate([dstf * ROI + srcf, dsts * ROI + srcs], axis=0)
    w_all = jnp.concatenate([wf, ws], axis=0)
    idx_all = jnp.pad(idx_all, ((0, 0), (0, _EPAD - LENN)))
    w_all = jnp.pad(w_all, ((0, 0), (0, _EPAD - LENN)))
    x3 = x.reshape(B, 2 * ROI, ROI - 1)
    xf3 = x3[:, :ROI]
    xs3 = x3[:, ROI:]
    b1r = b1.reshape(1, HID)
    b11r = b11.reshape(1, HID)
    b2r = b2.reshape(1, HID)
    w4r = w4.reshape(1, 2 * HID)
    b4r = b4.reshape(1, 1)
    bl1r, bl3r = bl1.reshape(1, -1), bl3.reshape(1, -1)
    bl11r, bl33r = bl11.reshape(1, -1), bl33.reshape(1, -1)
    bl4r, bl5r = bl4.reshape(1, -1), bl5.reshape(1, -1)
    bl6r, bl7r = bl6.reshape(1, -1), bl7.reshape(1, -1)

    A_flat = _sc_build(idx_all, w_all)
    Af = A_flat[:B].reshape(B, ROI, ROI)
    As = A_flat[B:].reshape(B, ROI, ROI)

    z1, z2, ap = pl.pallas_call(
        _z_body,
        grid=(B,),
        in_specs=[_g_spec(ROI, ROI - 1), _g_spec(ROI, ROI - 1),
                  _g_spec(ROI, ROI), _g_spec(ROI, ROI),
                  _w_spec((ROI - 1, HID)), _w_spec((1, HID)),
                  _w_spec((ROI - 1, HID)), _w_spec((1, HID)),
                  _w_spec((1, 2 * HID)), _w_spec((1, 1))],
        out_specs=[_g_spec(ROI, HID), _g_spec(ROI, HID), _g_spec(ROI, 1)],
        out_shape=[jax.ShapeDtypeStruct((B, ROI, HID), F32),
                   jax.ShapeDtypeStruct((B, ROI, HID), F32),
                   jax.ShapeDtypeStruct((B, ROI, 1), F32)],
    )(xf3, xs3, Af, As, W1, b1r, W11, b11r, w4r, b4r)

    z1f = z1.reshape(B, ROI * HID)
    z2f = z2.reshape(B, ROI * HID)
    apf = ap.reshape(B, ROI)
    x1, x2, alls = pl.pallas_call(
        _head_body,
        out_shape=[jax.ShapeDtypeStruct((B, 2), F32),
                   jax.ShapeDtypeStruct((B, 2), F32),
                   jax.ShapeDtypeStruct((B, ROI), F32)],
    )(z1f, z2f, apf, Wl1, bl1r, Wl3, bl3r, Wl11, bl11r, Wl33, bl33r,
      Wl4, bl4r, Wl5, bl5r)

    al3 = alls.reshape(B, ROI, 1)
    xc = pl.pallas_call(
        _g2_body,
        grid=(B,),
        in_specs=[_g_spec(ROI, HID), _g_spec(ROI, HID), _g_spec(ROI, 1),
                  _g_spec(ROI, ROI), _g_spec(ROI, ROI),
                  _w_spec((HID, HID)), _w_spec((1, HID))],
        out_specs=_g_spec(2 * ROI, HID),
        out_shape=jax.ShapeDtypeStruct((B, 2 * ROI, HID), F32),
    )(z1, z2, al3, Af, As, W2, b2r)

    xcf = xc.reshape(B, 2 * ROI * HID)
    xf = pl.pallas_call(
        _final_body,
        out_shape=jax.ShapeDtypeStruct((B, 2), F32),
    )(xcf, Wl6, bl6r, Wl7, bl7r)

    return (xf, x1, x2, alls)


# speedup vs baseline: 267.1557x; 1.5726x over previous
import functools

import jax
import jax.numpy as jnp
from jax import lax
from jax.experimental import pallas as pl
from jax.experimental.pallas import tpu as pltpu
from jax.experimental.pallas import tpu_sc as plsc

ROI = 116
LENN = 6670
B = 128
SEG2 = 2 * LENN + ROI
HID = 64
F32 = jnp.float32

_NC = 2
_NS = 16
_NW = _NC * _NS
_NMAT = 2 * B
_MPW = _NMAT // _NW
_EPAD = 6672
_AFLAT = ROI * ROI
_ACHUNKS = _AFLAT // 16
_ECHUNKS = _EPAD // 16

_DN = lambda c_lhs, c_rhs: ((c_lhs, c_rhs), ((), ()))


def _dot(a, b, dn=(((1,), (0,)), ((), ()))):
    return lax.dot_general(a, b, dn, preferred_element_type=F32)


def _build_body(srcf_ref, dstf_ref, wf_ref, srcs_ref, dsts_ref, ws_ref,
                af_ref, as_ref):
    n_iota = lax.broadcasted_iota(jnp.int32, (ROI, 1), 0)

    def one(src_ref, dst_ref, w_ref):
        src = src_ref[0]
        dst = dst_ref[0]
        w = w_ref[0]
        oh_s = (src == n_iota).astype(F32)
        oh_d = (dst == n_iota).astype(F32)
        return _dot(oh_d * w, oh_s, _DN((1,), (1,)))

    af_ref[0] = one(srcf_ref, dstf_ref, wf_ref)
    as_ref[0] = one(srcs_ref, dsts_ref, ws_ref)


def _sc_build_body(idx_hbm, w_hbm, out_hbm, idx_v, w_v, acc_v):
    wid = lax.axis_index("s") * _NC + lax.axis_index("c")

    def do_mat(i, carry):
        r = wid * _MPW + i
        pltpu.sync_copy(idx_hbm.at[r], idx_v)
        pltpu.sync_copy(w_hbm.at[r], w_v)

        def zero(j, c):
            acc_v[pl.ds(j * 16, 16)] = jnp.zeros((16,), F32)
            return c

        lax.fori_loop(0, _ACHUNKS, zero, 0, unroll=8)

        def scat(k, c):
            iv = idx_v[pl.ds(k * 16, 16)]
            wv = w_v[pl.ds(k * 16, 16)]
            plsc.addupdate_scatter(acc_v, [iv], wv)
            return c

        lax.fori_loop(0, _ECHUNKS, scat, 0, unroll=8)
        pltpu.sync_copy(acc_v, out_hbm.at[r])
        return carry

    lax.fori_loop(0, _MPW, do_mat, 0)


def _sc_build(idx_all, w_all):
    fn = pl.kernel(
        _sc_build_body,
        out_type=jax.ShapeDtypeStruct((_NMAT, _AFLAT), F32),
        mesh=plsc.VectorSubcoreMesh(core_axis_name="c", subcore_axis_name="s"),
        compiler_params=pltpu.CompilerParams(needs_layout_passes=False),
        scratch_types=[
            pltpu.VMEM((_EPAD,), jnp.int32),
            pltpu.VMEM((_EPAD,), F32),
            pltpu.VMEM((_AFLAT,), F32),
        ],
    )
    return fn(idx_all, w_all)


_G = 16
_BDOT = (((2,), (1,)), ((0,), (0,)))
_FDOT = (((2,), (0,)), ((), ()))


def _gcn_blk(x, A, W, b):
    h = lax.dot_general(x, W, _FDOT, preferred_element_type=F32)
    deg = jnp.sum(A, axis=2, keepdims=True) + 1.0
    dinv = lax.rsqrt(deg)
    m = lax.dot_general(A, dinv * h, _BDOT, preferred_element_type=F32)
    return jnp.maximum(dinv * m + (dinv * dinv) * h + b, 0.0)


def _z_body(xf_ref, xs_ref, af_ref, as_ref, W1_ref, b1_ref, W11_ref, b11_ref,
            w4_ref, b4_ref, z1_ref, z2_ref, ap_ref):
    z1 = _gcn_blk(xf_ref[...], af_ref[...], W1_ref[...], b1_ref[...])
    z2 = _gcn_blk(xs_ref[...], as_ref[...], W11_ref[...], b11_ref[...])
    z1_ref[...] = z1
    z2_ref[...] = z2
    w4 = w4_ref[...]
    ap = (jnp.sum(z1 * w4[:, :, :HID], axis=2, keepdims=True) +
          jnp.sum(z2 * w4[:, :, HID:], axis=2, keepdims=True) + b4_ref[0, 0, 0])
    ap_ref[...] = ap


def _softmax(l):
    m = jnp.max(l, axis=1, keepdims=True)
    e = jnp.exp(l - m)
    return e / jnp.sum(e, axis=1, keepdims=True)


def _head_body(z1f_ref, z2f_ref, ap_ref,
               Wl1_ref, bl1_ref, Wl3_ref, bl3_ref,
               Wl11_ref, bl11_ref, Wl33_ref, bl33_ref,
               Wl4_ref, bl4_ref, Wl5_ref, bl5_ref,
               x1_ref, x2_ref, alls_ref):
    def mlp_head(zf, Wa, ba, Wb, bb):
        t = jnp.maximum(_dot(zf, Wa) + ba, 0.0)
        return _softmax(_dot(t, Wb) + bb)

    x1_ref[...] = mlp_head(z1f_ref[...], Wl1_ref[...], bl1_ref[...],
                           Wl3_ref[...], bl3_ref[...])
    x2_ref[...] = mlp_head(z2f_ref[...], Wl11_ref[...], bl11_ref[...],
                           Wl33_ref[...], bl33_ref[...])
    a = jnp.maximum(_dot(ap_ref[...], Wl4_ref[...]) + bl4_ref[...], 0.0)
    a = jnp.maximum(_dot(a, Wl5_ref[...]) + bl5_ref[...], 0.0)
    alls_ref[...] = a


def _g2_body(z1_ref, z2_ref, al_ref, af_ref, as_ref, W2_ref, b2_ref, xc_ref):
    Af = af_ref[...]
    As = as_ref[...]
    al = al_ref[...]
    b2 = b2_ref[...]
    W2 = W2_ref[...]
    ht = lax.dot_general(z1_ref[...], W2, _FDOT, preferred_element_type=F32)
    hb = lax.dot_general(z2_ref[...], W2, _FDOT, preferred_element_type=F32)
    degt = jnp.sum(Af, axis=2, keepdims=True) + 1.0
    degb = jnp.sum(As, axis=2, keepdims=True) + al + 1.0
    dt = lax.rsqrt(degt)
    db = lax.rsqrt(degb)
    ot = dt * lax.dot_general(Af, dt * ht, _BDOT, preferred_element_type=F32) \
        + (dt * dt) * ht + b2
    ob = db * lax.dot_general(As, db * hb, _BDOT, preferred_element_type=F32) \
        + (db * db) * hb + (db * al * dt) * ht + b2
    xc_ref[...] = jnp.maximum(jnp.concatenate([ot, ob], axis=1), 0.0)


def _final_body(xcf_ref, Wl6_ref, bl6_ref, Wl7_ref, bl7_ref, xf_ref):
    t = jnp.maximum(_dot(xcf_ref[...], Wl6_ref[...]) + bl6_ref[...], 0.0)
    xf_ref[...] = _softmax(_dot(t, Wl7_ref[...]) + bl7_ref[...])


def _c_spec(*blk):
    return pl.BlockSpec((_G,) + blk, lambda g: (g,) + (0,) * len(blk))


def _w_spec(shape):
    nd = len(shape)
    return pl.BlockSpec(shape, lambda g: (0,) * nd)


def kernel(x, edge_index, edge_weight, roi_num, batch, device,
           W1, b1, W11, b11, W2, b2, w4, b4, Wl1, bl1, Wl3, bl3,
           Wl11, bl11, Wl33, bl33, Wl4, bl4, Wl5, bl5, Wl6, bl6, Wl7, bl7):
    del roi_num, batch, device
    ei = edge_index.astype(jnp.int32).reshape(2, B, SEG2)
    base = (jnp.arange(B, dtype=jnp.int32) * (2 * ROI))[None, :, None]
    loc = ei - base
    srcf = loc[0, :, :LENN]
    dstf = loc[1, :, :LENN]
    srcs = loc[0, :, LENN:2 * LENN] - ROI
    dsts = loc[1, :, LENN:2 * LENN] - ROI
    ew_r = edge_weight.reshape(B, SEG2)
    wf = ew_r[:, :LENN]
    ws = ew_r[:, LENN:2 * LENN]
    idx_all = jnp.concatenate([dstf * ROI + srcf, dsts * ROI + srcs], axis=0)
    w_all = jnp.concatenate([wf, ws], axis=0)
    idx_all = jnp.pad(idx_all, ((0, 0), (0, _EPAD - LENN)))
    w_all = jnp.pad(w_all, ((0, 0), (0, _EPAD - LENN)))
    x3 = x.reshape(B, 2 * ROI, ROI - 1)
    xf3 = x3[:, :ROI]
    xs3 = x3[:, ROI:]
    b1r = b1.reshape(1, 1, HID)
    b11r = b11.reshape(1, 1, HID)
    b2r = b2.reshape(1, 1, HID)
    w4r = w4.reshape(1, 1, 2 * HID)
    b4r = b4.reshape(1, 1, 1)
    bl1r, bl3r = bl1.reshape(1, -1), bl3.reshape(1, -1)
    bl11r, bl33r = bl11.reshape(1, -1), bl33.reshape(1, -1)
    bl4r, bl5r = bl4.reshape(1, -1), bl5.reshape(1, -1)
    bl6r, bl7r = bl6.reshape(1, -1), bl7.reshape(1, -1)

    A_flat = _sc_build(idx_all, w_all)
    Af = A_flat[:B].reshape(B, ROI, ROI)
    As = A_flat[B:].reshape(B, ROI, ROI)

    z1, z2, ap = pl.pallas_call(
        _z_body,
        grid=(B // _G,),
        in_specs=[_c_spec(ROI, ROI - 1), _c_spec(ROI, ROI - 1),
                  _c_spec(ROI, ROI), _c_spec(ROI, ROI),
                  _w_spec((ROI - 1, HID)), _w_spec((1, 1, HID)),
                  _w_spec((ROI - 1, HID)), _w_spec((1, 1, HID)),
                  _w_spec((1, 1, 2 * HID)), _w_spec((1, 1, 1))],
        out_specs=[_c_spec(ROI, HID), _c_spec(ROI, HID), _c_spec(ROI, 1)],
        out_shape=[jax.ShapeDtypeStruct((B, ROI, HID), F32),
                   jax.ShapeDtypeStruct((B, ROI, HID), F32),
                   jax.ShapeDtypeStruct((B, ROI, 1), F32)],
    )(xf3, xs3, Af, As, W1, b1r, W11, b11r, w4r, b4r)

    z1f = z1.reshape(B, ROI * HID)
    z2f = z2.reshape(B, ROI * HID)
    apf = ap.reshape(B, ROI)
    x1, x2, alls = pl.pallas_call(
        _head_body,
        out_shape=[jax.ShapeDtypeStruct((B, 2), F32),
                   jax.ShapeDtypeStruct((B, 2), F32),
                   jax.ShapeDtypeStruct((B, ROI), F32)],
    )(z1f, z2f, apf, Wl1, bl1r, Wl3, bl3r, Wl11, bl11r, Wl33, bl33r,
      Wl4, bl4r, Wl5, bl5r)

    al3 = alls.reshape(B, ROI, 1)
    xc = pl.pallas_call(
        _g2_body,
        grid=(B // _G,),
        in_specs=[_c_spec(ROI, HID), _c_spec(ROI, HID), _c_spec(ROI, 1),
                  _c_spec(ROI, ROI), _c_spec(ROI, ROI),
                  _w_spec((HID, HID)), _w_spec((1, 1, HID))],
        out_specs=_c_spec(2 * ROI, HID),
        out_shape=jax.ShapeDtypeStruct((B, 2 * ROI, HID), F32),
    )(z1, z2, al3, Af, As, W2, b2r)

    xcf = xc.reshape(B, 2 * ROI * HID)
    xf = pl.pallas_call(
        _final_body,
        out_shape=jax.ShapeDtypeStruct((B, 2), F32),
    )(xcf, Wl6, bl6r, Wl7, bl7r)

    return (xf, x1, x2, alls)


# speedup vs baseline: 526.1908x; 1.9696x over previous
import functools

import jax
import jax.numpy as jnp
from jax import lax
from jax.experimental import pallas as pl
from jax.experimental.pallas import tpu as pltpu
from jax.experimental.pallas import tpu_sc as plsc

ROI = 116
LENN = 6670
B = 128
SEG2 = 2 * LENN + ROI
HID = 64
F32 = jnp.float32

_NC = 2
_NS = 16
_NW = _NC * _NS
_GPW = B // _NW
_WLEN_FC = 6672
_WLEN_SC = 6688
_WSTART_SC = 6664
_LANE = 128
_PAD = 120

_DN = lambda c_lhs, c_rhs: ((c_lhs, c_rhs), ((), ()))


def _dot(a, b, dn=(((1,), (0,)), ((), ()))):
    return lax.dot_general(a, b, dn, preferred_element_type=F32)


def _sc_build_body(src_hbm, dst_hbm, ew_hbm, z_hbm, af_hbm, as_hbm,
                   src0, dst0, w0, acc0, src1, dst1, w1, acc1,
                   sem_in, sem_z, sem_out):
    wid = lax.axis_index("s") * _NC + lax.axis_index("c")
    g0 = wid * _GPW
    bufs = ((src0, dst0, w0, acc0), (src1, dst1, w1, acc1))

    def issue(t):
        br = t % 2
        g = g0 + t // 2
        start = g * SEG2 + (_WSTART_SC if br else 0)
        src_v, dst_v, w_v, acc_v = bufs[br]
        wlen = _WLEN_SC if br else _WLEN_FC
        return [
            pltpu.async_copy(src_hbm.at[pl.ds(start, wlen)], src_v, sem_in),
            pltpu.async_copy(dst_hbm.at[pl.ds(start, wlen)], dst_v, sem_in),
            pltpu.async_copy(ew_hbm.at[pl.ds(start, wlen)], w_v, sem_in),
            pltpu.async_copy(z_hbm, acc_v, sem_z),
        ]

    def scatter(t):
        br = t % 2
        g = g0 + t // 2
        off = g * (2 * ROI) + br * ROI
        src_v, dst_v, w_v, acc_v = bufs[br]
        lanes = lax.iota(jnp.int32, 16)

        def chunk(k, mask):
            sv = src_v[pl.ds(k * 16, 16)] - off
            dv = dst_v[pl.ds(k * 16, 16)] - off
            wv = w_v[pl.ds(k * 16, 16)]
            plsc.addupdate_scatter(acc_v, [dv, sv], wv, mask=mask)

        def body(k, c):
            chunk(k, None)
            return c

        if br == 0:
            lax.fori_loop(0, _WLEN_FC // 16 - 1, body, 0, unroll=8)
            chunk(_WLEN_FC // 16 - 1, lanes < 14)
        else:
            chunk(0, lanes >= 6)
            lax.fori_loop(1, _WLEN_SC // 16 - 1, body, 0, unroll=8)
            chunk(_WLEN_SC // 16 - 1, lanes < 4)

    in_d = {}
    out_d = {}
    in_d[0] = issue(0)
    for t in range(2 * _GPW):
        br = t % 2
        g = g0 + t // 2
        for d in in_d.pop(t):
            d.wait()
        if t + 1 < 2 * _GPW:
            if t - 1 >= 0:
                out_d[t - 1].wait()
            in_d[t + 1] = issue(t + 1)
        scatter(t)
        o_hbm = as_hbm if br else af_hbm
        out_d[t] = pltpu.async_copy(bufs[br][3],
                                    o_hbm.at[pl.ds(g * _PAD, _PAD)], sem_out)
    out_d[2 * _GPW - 2].wait()
    out_d[2 * _GPW - 1].wait()


def _sc_build(src1d, dst1d, ew1d, zmat):
    fn = pl.kernel(
        _sc_build_body,
        out_type=[jax.ShapeDtypeStruct((B * _PAD, _LANE), F32),
                  jax.ShapeDtypeStruct((B * _PAD, _LANE), F32)],
        mesh=plsc.VectorSubcoreMesh(core_axis_name="c", subcore_axis_name="s"),
        compiler_params=pltpu.CompilerParams(needs_layout_passes=False),
        scratch_types=[
            pltpu.VMEM((_WLEN_FC,), jnp.int32),
            pltpu.VMEM((_WLEN_FC,), jnp.int32),
            pltpu.VMEM((_WLEN_FC,), F32),
            pltpu.VMEM((_PAD, _LANE), F32),
            pltpu.VMEM((_WLEN_SC,), jnp.int32),
            pltpu.VMEM((_WLEN_SC,), jnp.int32),
            pltpu.VMEM((_WLEN_SC,), F32),
            pltpu.VMEM((_PAD, _LANE), F32),
            pltpu.SemaphoreType.DMA,
            pltpu.SemaphoreType.DMA,
            pltpu.SemaphoreType.DMA,
        ],
    )
    return fn(src1d, dst1d, ew1d, zmat)


_G = 16
_BDOT = (((2,), (1,)), ((0,), (0,)))
_FDOT = (((2,), (0,)), ((), ()))


def _gcn_blk(x, A, W, b):
    h = lax.dot_general(x, W, _FDOT, preferred_element_type=F32)
    deg = jnp.sum(A, axis=2, keepdims=True) + 1.0
    dinv = lax.rsqrt(deg)
    m = lax.dot_general(A, dinv * h, _BDOT, preferred_element_type=F32)
    return jnp.maximum(dinv * m + (dinv * dinv) * h + b, 0.0)


def _z_body(x_ref, af_ref, as_ref, W1_ref, b1_ref, W11_ref, b11_ref,
            w4_ref, b4_ref, z1_ref, z2_ref, ap_ref):
    xg = x_ref[...]
    Af = af_ref[...].reshape(_G, _PAD, _LANE)[:, :ROI, :ROI]
    As = as_ref[...].reshape(_G, _PAD, _LANE)[:, :ROI, :ROI]
    z1 = _gcn_blk(xg[:, :ROI, :], Af, W1_ref[...], b1_ref[...])
    z2 = _gcn_blk(xg[:, ROI:, :], As, W11_ref[...], b11_ref[...])
    z1_ref[...] = z1.reshape(_G, ROI * HID)
    z2_ref[...] = z2.reshape(_G, ROI * HID)
    w4 = w4_ref[...]
    ap = (jnp.sum(z1 * w4[:, :, :HID], axis=2) +
          jnp.sum(z2 * w4[:, :, HID:], axis=2) + b4_ref[0, 0, 0])
    ap_ref[...] = ap


def _softmax(l):
    m = jnp.max(l, axis=1, keepdims=True)
    e = jnp.exp(l - m)
    return e / jnp.sum(e, axis=1, keepdims=True)


def _head_body(z1f_ref, z2f_ref, ap_ref,
               Wl1_ref, bl1_ref, Wl3_ref, bl3_ref,
               Wl11_ref, bl11_ref, Wl33_ref, bl33_ref,
               Wl4_ref, bl4_ref, Wl5_ref, bl5_ref,
               x1_ref, x2_ref, alls_ref):
    def mlp_head(zf, Wa, ba, Wb, bb):
        t = jnp.maximum(_dot(zf, Wa) + ba, 0.0)
        return _softmax(_dot(t, Wb) + bb)

    x1_ref[...] = mlp_head(z1f_ref[...], Wl1_ref[...], bl1_ref[...],
                           Wl3_ref[...], bl3_ref[...])
    x2_ref[...] = mlp_head(z2f_ref[...], Wl11_ref[...], bl11_ref[...],
                           Wl33_ref[...], bl33_ref[...])
    a = jnp.maximum(_dot(ap_ref[...], Wl4_ref[...]) + bl4_ref[...], 0.0)
    a = jnp.maximum(_dot(a, Wl5_ref[...]) + bl5_ref[...], 0.0)
    alls_ref[...] = a


def _g2_body(z1_ref, z2_ref, al_ref, af_ref, as_ref, W2_ref, b2_ref, xc_ref):
    Af = af_ref[...].reshape(_G, _PAD, _LANE)[:, :ROI, :ROI]
    As = as_ref[...].reshape(_G, _PAD, _LANE)[:, :ROI, :ROI]
    al = al_ref[...].reshape(_G, ROI, 1)
    b2 = b2_ref[...]
    W2 = W2_ref[...]
    zt = z1_ref[...].reshape(_G, ROI, HID)
    zb = z2_ref[...].reshape(_G, ROI, HID)
    ht = lax.dot_general(zt, W2, _FDOT, preferred_element_type=F32)
    hb = lax.dot_general(zb, W2, _FDOT, preferred_element_type=F32)
    degt = jnp.sum(Af, axis=2, keepdims=True) + 1.0
    degb = jnp.sum(As, axis=2, keepdims=True) + al + 1.0
    dt = lax.rsqrt(degt)
    db = lax.rsqrt(degb)
    ot = dt * lax.dot_general(Af, dt * ht, _BDOT, preferred_element_type=F32) \
        + (dt * dt) * ht + b2
    ob = db * lax.dot_general(As, db * hb, _BDOT, preferred_element_type=F32) \
        + (db * db) * hb + (db * al * dt) * ht + b2
    xc = jnp.maximum(jnp.concatenate([ot, ob], axis=1), 0.0)
    xc_ref[...] = xc.reshape(_G, 2 * ROI * HID)


def _final_body(xcf_ref, Wl6_ref, bl6_ref, Wl7_ref, bl7_ref, xf_ref):
    t = jnp.maximum(_dot(xcf_ref[...], Wl6_ref[...]) + bl6_ref[...], 0.0)
    xf_ref[...] = _softmax(_dot(t, Wl7_ref[...]) + bl7_ref[...])


def _c_spec(*blk):
    return pl.BlockSpec((_G,) + blk, lambda g: (g,) + (0,) * len(blk))


_A_SPEC = pl.BlockSpec((_G * _PAD, _LANE), lambda g: (g, 0))


def _w_spec(shape):
    nd = len(shape)
    return pl.BlockSpec(shape, lambda g: (0,) * nd)


def kernel(x, edge_index, edge_weight, roi_num, batch, device,
           W1, b1, W11, b11, W2, b2, w4, b4, Wl1, bl1, Wl3, bl3,
           Wl11, bl11, Wl33, bl33, Wl4, bl4, Wl5, bl5, Wl6, bl6, Wl7, bl7):
    del roi_num, batch, device
    ei = edge_index.astype(jnp.int32)
    src1d = ei[0]
    dst1d = ei[1]
    zmat = jnp.zeros((_PAD, _LANE), F32)
    x3 = x.reshape(B, 2 * ROI, ROI - 1)
    b1r = b1.reshape(1, 1, HID)
    b11r = b11.reshape(1, 1, HID)
    b2r = b2.reshape(1, 1, HID)
    w4r = w4.reshape(1, 1, 2 * HID)
    b4r = b4.reshape(1, 1, 1)
    bl1r, bl3r = bl1.reshape(1, -1), bl3.reshape(1, -1)
    bl11r, bl33r = bl11.reshape(1, -1), bl33.reshape(1, -1)
    bl4r, bl5r = bl4.reshape(1, -1), bl5.reshape(1, -1)
    bl6r, bl7r = bl6.reshape(1, -1), bl7.reshape(1, -1)

    Af, As = _sc_build(src1d, dst1d, edge_weight, zmat)

    z1f, z2f, apf = pl.pallas_call(
        _z_body,
        grid=(B // _G,),
        in_specs=[_c_spec(2 * ROI, ROI - 1),
                  _A_SPEC, _A_SPEC,
                  _w_spec((ROI - 1, HID)), _w_spec((1, 1, HID)),
                  _w_spec((ROI - 1, HID)), _w_spec((1, 1, HID)),
                  _w_spec((1, 1, 2 * HID)), _w_spec((1, 1, 1))],
        out_specs=[_c_spec(ROI * HID), _c_spec(ROI * HID), _c_spec(ROI)],
        out_shape=[jax.ShapeDtypeStruct((B, ROI * HID), F32),
                   jax.ShapeDtypeStruct((B, ROI * HID), F32),
                   jax.ShapeDtypeStruct((B, ROI), F32)],
    )(x3, Af, As, W1, b1r, W11, b11r, w4r, b4r)

    x1, x2, alls = pl.pallas_call(
        _head_body,
        out_shape=[jax.ShapeDtypeStruct((B, 2), F32),
                   jax.ShapeDtypeStruct((B, 2), F32),
                   jax.ShapeDtypeStruct((B, ROI), F32)],
    )(z1f, z2f, apf, Wl1, bl1r, Wl3, bl3r, Wl11, bl11r, Wl33, bl33r,
      Wl4, bl4r, Wl5, bl5r)

    xcf = pl.pallas_call(
        _g2_body,
        grid=(B // _G,),
        in_specs=[_c_spec(ROI * HID), _c_spec(ROI * HID), _c_spec(ROI),
                  _A_SPEC, _A_SPEC,
                  _w_spec((HID, HID)), _w_spec((1, 1, HID))],
        out_specs=_c_spec(2 * ROI * HID),
        out_shape=jax.ShapeDtypeStruct((B, 2 * ROI * HID), F32),
    )(z1f, z2f, alls, Af, As, W2, b2r)

    xf = pl.pallas_call(
        _final_body,
        out_shape=jax.ShapeDtypeStruct((B, 2), F32),
    )(xcf, Wl6, bl6r, Wl7, bl7r)

    return (xf, x1, x2, alls)
